# baseline (speedup 1.0000x reference)
import functools

import jax
import jax.numpy as jnp
from jax import lax
from jax.experimental import pallas as pl
from jax.experimental.pallas import tpu as pltpu
from jax.experimental.pallas import tpu_sc as plsc

_N = 10000
_R = 4
_H = 4
_EMB = 128
_S = _EMB // _H
_E = 320000
_NRN = _N * _R
_BN = 400
_NB = _N // _BN
_BE = 2560

_NC = 2
_NS = 16
_NW = _NC * _NS
_C = 128
_NCHUNK = _E // _C
_GMAX = -(-_NCHUNK // _NW)
_ROWS_PER_TILE = 640
_N_PAD = _ROWS_PER_TILE * _NS
_FB = 64


def _blockdiag_t(mats):
    eye = jnp.eye(_H, dtype=mats.dtype)
    t = mats.transpose(0, 1, 3, 2)
    out6 = eye[None, :, :, None, None] * t[:, :, None, :, :]
    return out6.transpose(0, 1, 3, 2, 4).reshape(_R, _EMB, _EMB)


def _proj_body(x_ref, wk_ref, wq_ref, wv_ref, k_ref, q_ref, v_ref):
    x = x_ref[...]
    k_ref[...] = jnp.dot(x, wk_ref[0], preferred_element_type=jnp.float32)
    q_ref[...] = jnp.dot(x, wq_ref[0], preferred_element_type=jnp.float32)
    v_ref[...] = jnp.dot(x, wv_ref[0], preferred_element_type=jnp.float32)


def _project_tables(x, wk, wq, wv):
    wspec = pl.BlockSpec((1, _EMB, _EMB), lambda p, i: (p, 0, 0))
    ospec = pl.BlockSpec((_BN, _EMB), lambda p, i: (p * _NB + i, 0))
    oshape = jax.ShapeDtypeStruct((_R * _N, _EMB), jnp.float32)
    return pl.pallas_call(
        _proj_body,
        grid=(_R, _NB),
        in_specs=[
            pl.BlockSpec((_BN, _EMB), lambda p, i: (i, 0)),
            wspec, wspec, wspec,
        ],
        out_specs=[ospec, ospec, ospec],
        out_shape=[oshape, oshape, oshape],
    )(x, wk, wq, wv)


def _final_body(a_ref, b_ref, w_ref, o_ref):
    o_ref[...] = jnp.dot(a_ref[...] + b_ref[...], w_ref[...],
                         preferred_element_type=jnp.float32)


def _final_matmul(agg0, agg1, w_unify):
    return pl.pallas_call(
        _final_body,
        grid=(_NB,),
        in_specs=[
            pl.BlockSpec((_BN, _EMB), lambda i: (i, 0)),
            pl.BlockSpec((_BN, _EMB), lambda i: (i, 0)),
            pl.BlockSpec((_EMB, _EMB), lambda i: (0, 0)),
        ],
        out_specs=pl.BlockSpec((_BN, _EMB), lambda i: (i, 0)),
        out_shape=jax.ShapeDtypeStruct((_N, _EMB), jnp.float32),
    )(agg0, agg1, w_unify.T)


_SC_MESH = plsc.VectorSubcoreMesh(
    core_axis_name="c", subcore_axis_name="s", num_cores=_NC, num_subcores=_NS)


@functools.partial(
    pl.kernel,
    out_type=jax.ShapeDtypeStruct((_E, _EMB), jnp.float32),
    mesh=_SC_MESH,
    scratch_types=[
        pltpu.VMEM((_C,), jnp.int32),
        pltpu.VMEM((_C,), jnp.int32),
        pltpu.VMEM((_C, _EMB), jnp.float32),
        pltpu.VMEM((_C, _EMB), jnp.float32),
        pltpu.SemaphoreType.DMA,
        pltpu.SemaphoreType.DMA,
    ],
)
def _sc_gather(kt_hbm, qt_hbm, row_hbm, col_hbm, prod_hbm,
               rowi, coli, kbuf, qbuf, sem0, sem1):
    wid = lax.axis_index("s") * _NC + lax.axis_index("c")

    def chunk(g, _):
        cid = g * _NW + wid

        @pl.when(cid < _NCHUNK)
        def _():
            base = cid * _C
            pltpu.sync_copy(row_hbm.at[pl.ds(base, _C)], rowi)
            pltpu.sync_copy(col_hbm.at[pl.ds(base, _C)], coli)
            ck = pltpu.async_copy(kt_hbm.at[rowi], kbuf, sem0)
            cq = pltpu.async_copy(qt_hbm.at[coli], qbuf, sem1)
            ck.wait()
            cq.wait()

            def erow(e, _):
                for k in range(_EMB // 16):
                    kbuf[e, pl.ds(k * 16, 16)] = (
                        kbuf[e, pl.ds(k * 16, 16)] * qbuf[e, pl.ds(k * 16, 16)])
                return _

            lax.fori_loop(0, _C, erow, None)
            pltpu.sync_copy(kbuf, prod_hbm.at[pl.ds(base, _C)])
        return _

    lax.fori_loop(0, _GMAX, chunk, None)


@functools.partial(
    pl.kernel,
    out_type=jax.ShapeDtypeStruct((_NC * _N_PAD, _EMB), jnp.float32),
    mesh=_SC_MESH,
    scratch_types=[
        pltpu.VMEM((_C,), jnp.int32),
        pltpu.VMEM((_C,), jnp.int32),
        pltpu.VMEM((_C,), jnp.int32),
        pltpu.VMEM((_C // 4, 16), jnp.float32),
        pltpu.VMEM((_C, _EMB), jnp.float32),
        pltpu.VMEM((_C, _EMB), jnp.float32),
        pltpu.VMEM((_FB, _EMB), jnp.float32),
        pltpu.VMEM_SHARED((_N_PAD, _EMB), jnp.float32),
        pltpu.SemaphoreType.DMA,
        pltpu.SemaphoreType.DMA,
    ],
)
def _sc_spmm(dot_hbm, recip_hbm, vt_hbm, sub_hbm, row_hbm, col_hbm, agg_hbm,
             subi, rowi, coli, dotb, rbuf, vbuf, zbuf, agg_sh, sem, sem1):
    cidx = lax.axis_index("c")
    sidx = lax.axis_index("s")
    wid = sidx * _NC + cidx

    zvec = jnp.zeros((16,), jnp.float32)

    def zrow(i, _):
        for k in range(_EMB // 16):
            zbuf[i, pl.ds(k * 16, 16)] = zvec
        return _

    lax.fori_loop(0, _FB, zrow, None)

    def zblk(b, _):
        lo = sidx * _ROWS_PER_TILE + b * _FB
        pltpu.sync_copy(zbuf, agg_sh.at[pl.ds(lo, _FB)])
        return _

    lax.fori_loop(0, _ROWS_PER_TILE // _FB, zblk, None)
    plsc.subcore_barrier()

    def chunk(g, _):
        cid = g * _NW + wid

        @pl.when(cid < _NCHUNK)
        def _():
            base = cid * _C
            pltpu.sync_copy(sub_hbm.at[pl.ds(base, _C)], subi)
            pltpu.sync_copy(row_hbm.at[pl.ds(base, _C)], rowi)
            pltpu.sync_copy(col_hbm.at[pl.ds(base, _C)], coli)
            pltpu.sync_copy(dot_hbm.at[pl.ds(cid * (_C // 4), _C // 4)], dotb)
            cr = pltpu.async_copy(recip_hbm.at[rowi], rbuf, sem1)
            cv = pltpu.async_copy(vt_hbm.at[coli], vbuf, sem)
            cr.wait()
            cv.wait()

            def edge4(j, _):
                dv = dotb[j, pl.ds(0, 16)]
                for t in range(4):
                    e = j * 4 + t
                    rv = rbuf[e, pl.ds(0, 16)]
                    for h in range(_H):
                        s = dv[4 * t + h] * rv[h]
                        bc = jnp.full((16,), s, jnp.float32)
                        for k in (2 * h, 2 * h + 1):
                            vbuf[e, pl.ds(k * 16, 16)] = (
                                vbuf[e, pl.ds(k * 16, 16)] * bc)
                return _

            lax.fori_loop(0, _C // 4, edge4, None)
            pltpu.sync_copy(vbuf, agg_sh.at[subi], add=True)
        return _

    lax.fori_loop(0, _GMAX, chunk, None)
    plsc.subcore_barrier()

    def fblk(b, _):
        lo = sidx * _ROWS_PER_TILE + b * _FB
        pltpu.sync_copy(agg_sh.at[pl.ds(lo, _FB)], zbuf)
        pltpu.sync_copy(zbuf, agg_hbm.at[pl.ds(cidx * _N_PAD + lo, _FB)])
        return _

    lax.fori_loop(0, _ROWS_PER_TILE // _FB, fblk, None)


def kernel(x, tokeys, toqueries, tovals, W_unify, indices, mindices):
    wk = _blockdiag_t(tokeys)
    wq = _blockdiag_t(toqueries)
    wv = _blockdiag_t(tovals)
    kt, qt, vt = _project_tables(x, wk, wq, wv)

    sub = indices[:, 0]
    row = mindices[:, 0]
    col = mindices[:, 1]

    prod = _sc_gather(kt, qt, row, col)

    dot = prod.reshape(_E, _H, _S).sum(-1).T
    rowsum = jax.vmap(
        lambda dd: jax.ops.segment_sum(dd, row, num_segments=_NRN))(dot)
    recip128 = jnp.pad((1.0 / rowsum).T, ((0, 0), (0, _EMB - _H)))

    agg = _sc_spmm(dot.T.reshape(_E // 4, 16), recip128, vt, sub, row, col)

    return _final_matmul(agg[:_N], agg[_N_PAD:_N_PAD + _N], W_unify)


# speedup vs baseline: 1.0976x; 1.0976x over previous
import functools

import jax
import jax.numpy as jnp
from jax import lax
from jax.experimental import pallas as pl
from jax.experimental.pallas import tpu as pltpu
from jax.experimental.pallas import tpu_sc as plsc

_N = 10000
_R = 4
_H = 4
_EMB = 128
_S = _EMB // _H
_E = 320000
_NRN = _N * _R
_BN = 400
_NB = _N // _BN

_NC = 2
_NS = 16
_NW = _NC * _NS
_C = 128
_E2 = _E // 2
_NCHUNK = _E2 // _C
_GMAX = -(-_NCHUNK // _NW)
_ROWS_PER_TILE = 640
_N_PAD = _ROWS_PER_TILE * _NS
_FB = 64


def _blockdiag_t(mats):
    eye = jnp.eye(_H, dtype=mats.dtype)
    t = mats.transpose(0, 1, 3, 2)
    out6 = eye[None, :, :, None, None] * t[:, :, None, :, :]
    return out6.transpose(0, 1, 3, 2, 4).reshape(_R, _EMB, _EMB)


def _proj_body(x_ref, wk_ref, wq_ref, wv_ref, k_ref, q_ref, v_ref):
    x = x_ref[...]
    k_ref[...] = jnp.dot(x, wk_ref[0], preferred_element_type=jnp.float32)
    q_ref[...] = jnp.dot(x, wq_ref[0], preferred_element_type=jnp.float32)
    v_ref[...] = jnp.dot(x, wv_ref[0], preferred_element_type=jnp.float32)


def _project_tables(x, wk, wq, wv):
    wspec = pl.BlockSpec((1, _EMB, _EMB), lambda p, i: (p, 0, 0))
    ospec = pl.BlockSpec((_BN, _EMB), lambda p, i: (p * _NB + i, 0))
    oshape = jax.ShapeDtypeStruct((_R * _N, _EMB), jnp.float32)
    return pl.pallas_call(
        _proj_body,
        grid=(_R, _NB),
        in_specs=[
            pl.BlockSpec((_BN, _EMB), lambda p, i: (i, 0)),
            wspec, wspec, wspec,
        ],
        out_specs=[ospec, ospec, ospec],
        out_shape=[oshape, oshape, oshape],
    )(x, wk, wq, wv)


def _final_body(a_ref, b_ref, c_ref, d_ref, w_ref, o_ref):
    s = (a_ref[...] + b_ref[...]) + (c_ref[...] + d_ref[...])
    o_ref[...] = jnp.dot(s, w_ref[...], preferred_element_type=jnp.float32)


def _final_matmul(agg0, agg1, agg2, agg3, w_unify):
    aspec = pl.BlockSpec((_BN, _EMB), lambda i: (i, 0))
    return pl.pallas_call(
        _final_body,
        grid=(_NB,),
        in_specs=[
            aspec, aspec, aspec, aspec,
            pl.BlockSpec((_EMB, _EMB), lambda i: (0, 0)),
        ],
        out_specs=aspec,
        out_shape=jax.ShapeDtypeStruct((_N, _EMB), jnp.float32),
    )(agg0, agg1, agg2, agg3, w_unify.T)


_SC_MESH = plsc.VectorSubcoreMesh(
    core_axis_name="c", subcore_axis_name="s", num_cores=_NC, num_subcores=_NS)


@functools.partial(
    pl.kernel,
    out_type=jax.ShapeDtypeStruct((_E2, _EMB), jnp.float32),
    mesh=_SC_MESH,
    scratch_types=[
        pltpu.VMEM((_C,), jnp.int32),
        pltpu.VMEM((_C,), jnp.int32),
        pltpu.VMEM((_C, _EMB), jnp.float32),
        pltpu.VMEM((_C, _EMB), jnp.float32),
        pltpu.SemaphoreType.DMA,
        pltpu.SemaphoreType.DMA,
    ],
)
def _sc_gather(kt_hbm, qt_hbm, row_hbm, col_hbm, prod_hbm,
               rowi, coli, kbuf, qbuf, sem0, sem1):
    wid = lax.axis_index("s") * _NC + lax.axis_index("c")

    def chunk(g, _):
        cid = g * _NW + wid

        @pl.when(cid < _NCHUNK)
        def _():
            base = cid * _C
            pltpu.sync_copy(row_hbm.at[pl.ds(base, _C)], rowi)
            pltpu.sync_copy(col_hbm.at[pl.ds(base, _C)], coli)
            ck = pltpu.async_copy(kt_hbm.at[rowi], kbuf, sem0)
            cq = pltpu.async_copy(qt_hbm.at[coli], qbuf, sem1)
            ck.wait()
            cq.wait()

            def erow(e, _):
                for k in range(_EMB // 16):
                    kbuf[e, pl.ds(k * 16, 16)] = (
                        kbuf[e, pl.ds(k * 16, 16)] * qbuf[e, pl.ds(k * 16, 16)])
                return _

            lax.fori_loop(0, _C, erow, None)
            pltpu.sync_copy(kbuf, prod_hbm.at[pl.ds(base, _C)])
        return _

    lax.fori_loop(0, _GMAX, chunk, None)


@functools.partial(
    pl.kernel,
    out_type=jax.ShapeDtypeStruct((_NC * _N_PAD, _EMB), jnp.float32),
    mesh=_SC_MESH,
    scratch_types=[
        pltpu.VMEM((_C,), jnp.int32),
        pltpu.VMEM((_C,), jnp.int32),
        pltpu.VMEM((_C,), jnp.int32),
        pltpu.VMEM((_C // 4, 16), jnp.float32),
        pltpu.VMEM((_C, _EMB), jnp.float32),
        pltpu.VMEM((_C, _EMB), jnp.float32),
        pltpu.VMEM((_FB, _EMB), jnp.float32),
        pltpu.VMEM_SHARED((_N_PAD, _EMB), jnp.float32),
        pltpu.SemaphoreType.DMA,
        pltpu.SemaphoreType.DMA,
    ],
)
def _sc_spmm(dot_hbm, recip_hbm, vt_hbm, sub_hbm, row_hbm, col_hbm, agg_hbm,
             subi, rowi, coli, dotb, rbuf, vbuf, zbuf, agg_sh, sem, sem1):
    cidx = lax.axis_index("c")
    sidx = lax.axis_index("s")
    wid = sidx * _NC + cidx

    zvec = jnp.zeros((16,), jnp.float32)

    def zrow(i, _):
        for k in range(_EMB // 16):
            zbuf[i, pl.ds(k * 16, 16)] = zvec
        return _

    lax.fori_loop(0, _FB, zrow, None)

    def zblk(b, _):
        lo = sidx * _ROWS_PER_TILE + b * _FB
        pltpu.sync_copy(zbuf, agg_sh.at[pl.ds(lo, _FB)])
        return _

    lax.fori_loop(0, _ROWS_PER_TILE // _FB, zblk, None)
    plsc.subcore_barrier()

    def chunk(g, _):
        cid = g * _NW + wid

        @pl.when(cid < _NCHUNK)
        def _():
            base = cid * _C
            pltpu.sync_copy(sub_hbm.at[pl.ds(base, _C)], subi)
            pltpu.sync_copy(row_hbm.at[pl.ds(base, _C)], rowi)
            pltpu.sync_copy(col_hbm.at[pl.ds(base, _C)], coli)
            pltpu.sync_copy(dot_hbm.at[pl.ds(cid * (_C // 4), _C // 4)], dotb)
            cr = pltpu.async_copy(recip_hbm.at[rowi], rbuf, sem1)
            cv = pltpu.async_copy(vt_hbm.at[coli], vbuf, sem)
            cr.wait()
            cv.wait()

            def edge4(j, _):
                dv = dotb[j, pl.ds(0, 16)]
                for t in range(4):
                    e = j * 4 + t
                    rv = rbuf[e, pl.ds(0, 16)]
                    for h in range(_H):
                        s = dv[4 * t + h] * rv[h]
                        bc = jnp.full((16,), s, jnp.float32)
                        for k in (2 * h, 2 * h + 1):
                            vbuf[e, pl.ds(k * 16, 16)] = (
                                vbuf[e, pl.ds(k * 16, 16)] * bc)
                return _

            lax.fori_loop(0, _C // 4, edge4, None)
            pltpu.sync_copy(vbuf, agg_sh.at[subi], add=True)
        return _

    lax.fori_loop(0, _GMAX, chunk, None)
    plsc.subcore_barrier()

    def fblk(b, _):
        lo = sidx * _ROWS_PER_TILE + b * _FB
        pltpu.sync_copy(agg_sh.at[pl.ds(lo, _FB)], zbuf)
        pltpu.sync_copy(zbuf, agg_hbm.at[pl.ds(cidx * _N_PAD + lo, _FB)])
        return _

    lax.fori_loop(0, _ROWS_PER_TILE // _FB, fblk, None)


def kernel(x, tokeys, toqueries, tovals, W_unify, indices, mindices):
    wk = _blockdiag_t(tokeys)
    wq = _blockdiag_t(toqueries)
    wv = _blockdiag_t(tovals)
    kt, qt, vt = _project_tables(x, wk, wq, wv)

    sub = indices[:, 0]
    row = mindices[:, 0]
    col = mindices[:, 1]

    aggs = []
    for half in range(2):
        sl = slice(half * _E2, (half + 1) * _E2)
        rowh, colh, subh = row[sl], col[sl], sub[sl]
        prod = _sc_gather(kt, qt, rowh, colh)
        dot = prod.reshape(_E2, _H, _S).sum(-1).T
        rowsum = jax.vmap(
            lambda dd: jax.ops.segment_sum(dd, rowh, num_segments=_NRN))(dot)
        recip128 = jnp.pad((1.0 / rowsum).T, ((0, 0), (0, _EMB - _H)))
        aggs.append(_sc_spmm(dot.T.reshape(_E2 // 4, 16), recip128, vt,
                             subh, rowh, colh))

    return _final_matmul(aggs[0][:_N], aggs[0][_N_PAD:_N_PAD + _N],
                         aggs[1][:_N], aggs[1][_N_PAD:_N_PAD + _N], W_unify)


# speedup vs baseline: 1.2100x; 1.1023x over previous
import functools

import jax
import jax.numpy as jnp
from jax import lax
from jax.experimental import pallas as pl
from jax.experimental.pallas import tpu as pltpu
from jax.experimental.pallas import tpu_sc as plsc

_N = 10000
_R = 4
_H = 4
_EMB = 128
_S = _EMB // _H
_E = 320000
_NRN = _N * _R
_BN = 400
_NB = _N // _BN

_NC = 2
_NS = 16
_NW = _NC * _NS
_C = 128
_E2 = _E // 2
_NCHUNK = _E2 // _C
_GMAX = -(-_NCHUNK // _NW)
_PW = _GMAX * _C
_ROWS_PER_TILE = 640
_N_PAD = _ROWS_PER_TILE * _NS
_FB = _C


def _blockdiag_t(mats):
    eye = jnp.eye(_H, dtype=mats.dtype)
    t = mats.transpose(0, 1, 3, 2)
    out6 = eye[None, :, :, None, None] * t[:, :, None, :, :]
    return out6.transpose(0, 1, 3, 2, 4).reshape(_R, _EMB, _EMB)


def _proj_body(x_ref, wk_ref, wq_ref, wv_ref, k_ref, q_ref, v_ref):
    x = x_ref[...]
    k_ref[...] = jnp.dot(x, wk_ref[0], preferred_element_type=jnp.float32)
    q_ref[...] = jnp.dot(x, wq_ref[0], preferred_element_type=jnp.float32)
    v_ref[...] = jnp.dot(x, wv_ref[0], preferred_element_type=jnp.float32)


def _project_tables(x, wk, wq, wv):
    wspec = pl.BlockSpec((1, _EMB, _EMB), lambda p, i: (p, 0, 0))
    ospec = pl.BlockSpec((_BN, _EMB), lambda p, i: (p * _NB + i, 0))
    oshape = jax.ShapeDtypeStruct((_R * _N, _EMB), jnp.float32)
    return pl.pallas_call(
        _proj_body,
        grid=(_R, _NB),
        in_specs=[
            pl.BlockSpec((_BN, _EMB), lambda p, i: (i, 0)),
            wspec, wspec, wspec,
        ],
        out_specs=[ospec, ospec, ospec],
        out_shape=[oshape, oshape, oshape],
    )(x, wk, wq, wv)


def _final_body(a_ref, b_ref, c_ref, d_ref, w_ref, o_ref):
    s = (a_ref[...] + b_ref[...]) + (c_ref[...] + d_ref[...])
    o_ref[...] = jnp.dot(s, w_ref[...], preferred_element_type=jnp.float32)


def _final_matmul(agg0, agg1, agg2, agg3, w_unify):
    aspec = pl.BlockSpec((_BN, _EMB), lambda i: (i, 0))
    return pl.pallas_call(
        _final_body,
        grid=(_NB,),
        in_specs=[
            aspec, aspec, aspec, aspec,
            pl.BlockSpec((_EMB, _EMB), lambda i: (0, 0)),
        ],
        out_specs=aspec,
        out_shape=jax.ShapeDtypeStruct((_N, _EMB), jnp.float32),
    )(agg0, agg1, agg2, agg3, w_unify.T)


_SC_MESH = plsc.VectorSubcoreMesh(
    core_axis_name="c", subcore_axis_name="s", num_cores=_NC, num_subcores=_NS)


@functools.partial(
    pl.kernel,
    out_type=jax.ShapeDtypeStruct((_E2, _EMB), jnp.float32),
    mesh=_SC_MESH,
    scratch_types=[
        pltpu.VMEM((_PW,), jnp.int32),
        pltpu.VMEM((_PW,), jnp.int32),
        pltpu.VMEM((_C, _EMB), jnp.float32),
        pltpu.VMEM((_C, _EMB), jnp.float32),
        pltpu.SemaphoreType.DMA,
        pltpu.SemaphoreType.DMA,
    ],
)
def _sc_gather(kt_hbm, qt_hbm, row_hbm, col_hbm, prod_hbm,
               rowi, coli, kbuf, qbuf, sem0, sem1):
    wid = lax.axis_index("s") * _NC + lax.axis_index("c")
    start = wid * _NCHUNK // _NW
    count = (wid + 1) * _NCHUNK // _NW - start
    pltpu.sync_copy(row_hbm.at[pl.ds(start * _C, _PW)], rowi)
    pltpu.sync_copy(col_hbm.at[pl.ds(start * _C, _PW)], coli)

    def chunk(g, _):
        @pl.when(g < count)
        def _():
            base = (start + g) * _C
            ck = pltpu.async_copy(
                kt_hbm.at[rowi.at[pl.ds(g * _C, _C)]], kbuf, sem0)
            cq = pltpu.async_copy(
                qt_hbm.at[coli.at[pl.ds(g * _C, _C)]], qbuf, sem1)
            ck.wait()
            cq.wait()

            def erow(e, _):
                for k in range(_EMB // 16):
                    kbuf[e, pl.ds(k * 16, 16)] = (
                        kbuf[e, pl.ds(k * 16, 16)] * qbuf[e, pl.ds(k * 16, 16)])
                return _

            lax.fori_loop(0, _C, erow, None)
            pltpu.sync_copy(kbuf, prod_hbm.at[pl.ds(base, _C)])
        return _

    lax.fori_loop(0, _GMAX, chunk, None)


@functools.partial(
    pl.kernel,
    out_type=jax.ShapeDtypeStruct((_NC * _N_PAD, _EMB), jnp.float32),
    mesh=_SC_MESH,
    scratch_types=[
        pltpu.VMEM((_C,), jnp.int32),
        pltpu.VMEM((_PW,), jnp.int32),
        pltpu.VMEM((_PW,), jnp.int32),
        pltpu.VMEM((_C // 4, 16), jnp.float32),
        pltpu.VMEM((_C, _EMB), jnp.float32),
        pltpu.VMEM((_C, _EMB), jnp.float32),
        pltpu.VMEM_SHARED((_N_PAD, _EMB), jnp.float32),
        pltpu.SemaphoreType.DMA,
        pltpu.SemaphoreType.DMA,
    ],
)
def _sc_spmm(dot_hbm, recip_hbm, vt_hbm, sub_hbm, row_hbm, col_hbm, agg_hbm,
             subi, rowi, coli, dotb, rbuf, vbuf, agg_sh, sem, sem1):
    cidx = lax.axis_index("c")
    sidx = lax.axis_index("s")
    wid = sidx * _NC + cidx

    zvec = jnp.zeros((16,), jnp.float32)

    def zrow(i, _):
        for k in range(_EMB // 16):
            vbuf[i, pl.ds(k * 16, 16)] = zvec
        return _

    lax.fori_loop(0, _FB, zrow, None)

    def zblk(b, _):
        lo = sidx * _ROWS_PER_TILE + b * _FB
        pltpu.sync_copy(vbuf, agg_sh.at[pl.ds(lo, _FB)])
        return _

    lax.fori_loop(0, _ROWS_PER_TILE // _FB, zblk, None)
    start = wid * _NCHUNK // _NW
    count = (wid + 1) * _NCHUNK // _NW - start
    pltpu.sync_copy(row_hbm.at[pl.ds(start * _C, _PW)], rowi)
    pltpu.sync_copy(col_hbm.at[pl.ds(start * _C, _PW)], coli)
    plsc.subcore_barrier()

    def chunk(g, _):
        @pl.when(g < count)
        def _():
            cid = start + g
            base = cid * _C
            pltpu.sync_copy(dot_hbm.at[pl.ds(cid * (_C // 4), _C // 4)], dotb)
            cr = pltpu.async_copy(
                recip_hbm.at[rowi.at[pl.ds(g * _C, _C)]], rbuf, sem1)
            cv = pltpu.async_copy(
                vt_hbm.at[coli.at[pl.ds(g * _C, _C)]], vbuf, sem)
            pltpu.sync_copy(sub_hbm.at[pl.ds(base, _C)], subi)
            cr.wait()
            cv.wait()

            def edge4(j, _):
                dv = dotb[j, pl.ds(0, 16)]
                for t in range(4):
                    e = j * 4 + t
                    rv = rbuf[e, pl.ds(0, 16)]
                    for h in range(_H):
                        s = dv[4 * t + h] * rv[h]
                        bc = jnp.full((16,), s, jnp.float32)
                        for k in (2 * h, 2 * h + 1):
                            vbuf[e, pl.ds(k * 16, 16)] = (
                                vbuf[e, pl.ds(k * 16, 16)] * bc)
                return _

            lax.fori_loop(0, _C // 4, edge4, None)
            pltpu.sync_copy(vbuf, agg_sh.at[subi], add=True)
        return _

    lax.fori_loop(0, _GMAX, chunk, None)
    plsc.subcore_barrier()

    def fblk(b, _):
        lo = sidx * _ROWS_PER_TILE + b * _FB
        pltpu.sync_copy(agg_sh.at[pl.ds(lo, _FB)], vbuf)
        pltpu.sync_copy(vbuf, agg_hbm.at[pl.ds(cidx * _N_PAD + lo, _FB)])
        return _

    lax.fori_loop(0, _ROWS_PER_TILE // _FB, fblk, None)


def kernel(x, tokeys, toqueries, tovals, W_unify, indices, mindices):
    wk = _blockdiag_t(tokeys)
    wq = _blockdiag_t(toqueries)
    wv = _blockdiag_t(tovals)
    kt, qt, vt = _project_tables(x, wk, wq, wv)

    sub = indices[:, 0]
    row = mindices[:, 0]
    col = mindices[:, 1]

    aggs = []
    for half in range(2):
        sl = slice(half * _E2, (half + 1) * _E2)
        rowh, colh, subh = row[sl], col[sl], sub[sl]
        prod = _sc_gather(kt, qt, rowh, colh)
        dot = prod.reshape(_E2, _H, _S).sum(-1).T
        rowsum = jax.vmap(
            lambda dd: jax.ops.segment_sum(dd, rowh, num_segments=_NRN))(dot)
        recip128 = jnp.pad((1.0 / rowsum).T, ((0, 0), (0, _EMB - _H)))
        aggs.append(_sc_spmm(dot.T.reshape(_E2 // 4, 16), recip128, vt,
                             subh, rowh, colh))

    return _final_matmul(aggs[0][:_N], aggs[0][_N_PAD:_N_PAD + _N],
                         aggs[1][:_N], aggs[1][_N_PAD:_N_PAD + _N], W_unify)
